# baseline (speedup 1.0000x reference)
import functools

import jax
import jax.numpy as jnp
from jax import lax
from jax.experimental import pallas as pl
from jax.experimental.pallas import tpu as pltpu
from jax.experimental.pallas import tpu_sc as plsc

_NC = 2
_NS = 16
_NW = _NC * _NS
_L = 16
_CH = 128

_IDX0 = 124


def kernel(x):
    n, d = x.shape
    assert n % (_NW * _CH) == 0
    rows = n // _NW
    nch = rows // _CH
    mesh = plsc.VectorSubcoreMesh(core_axis_name="c", subcore_axis_name="s")

    @functools.partial(
        pl.kernel,
        out_type=jax.ShapeDtypeStruct((n * d,), x.dtype),
        mesh=mesh,
        scratch_types=[
            pltpu.VMEM((rows * d,), jnp.float32),
            pltpu.VMEM((4 * nch, _CH), jnp.int32),
            pltpu.VMEM((rows,), jnp.float32),
            pltpu.VMEM((rows,), jnp.float32),
            pltpu.VMEM((rows,), jnp.float32),
            pltpu.VMEM((rows,), jnp.float32),
            pltpu.SemaphoreType.DMA,
        ],
    )
    def run(x_hbm, out_hbm, slab, idx, bb, cb, lb, gb, sem):
        wid = lax.axis_index("s") * _NC + lax.axis_index("c")
        base = wid * rows
        flat_sl = pl.ds(base * d, rows * d)

        in_cp = pltpu.async_copy(x_hbm.at[flat_sl], slab, sem)

        lane = lax.iota(jnp.int32, _L)

        def build(k, carry):
            j = k // (nch * (_CH // _L))
            r = k % (nch * (_CH // _L))
            ch = r // (_CH // _L)
            v = r % (_CH // _L)
            row0 = base + ch * _CH + v * _L
            idx[j * nch + ch, pl.ds(v * _L, _L)] = (row0 + lane) * d + _IDX0 + j
            return carry

        lax.fori_loop(0, 4 * nch * (_CH // _L), build, 0)

        for j, buf in ((0, bb), (1, cb), (2, lb), (3, gb)):
            for ch in range(nch):
                pltpu.sync_copy(x_hbm.at[idx.at[j * nch + ch]],
                                buf.at[pl.ds(ch * _CH, _CH)])

        def step(k, carry):
            sl = pl.ds(k * _L, _L)
            b = bb[sl]
            c = cb[sl]
            l = lb[sl]
            g = gb[sl]
            cb[sl] = c / (c + b)
            lb[sl] = c / (c + l + g)
            gb[sl] = g / (g + l)
            return carry

        lax.fori_loop(0, rows // _L, step, 0)

        in_cp.wait()
        pltpu.sync_copy(slab, out_hbm.at[flat_sl])

        for j, buf in ((1, cb), (2, lb), (3, gb)):
            for ch in range(nch):
                pltpu.sync_copy(buf.at[pl.ds(ch * _CH, _CH)],
                                out_hbm.at[idx.at[j * nch + ch]])

    return run(x.reshape(-1)).reshape(n, d)


# speedup vs baseline: 1.1115x; 1.1115x over previous
import functools

import jax
import jax.numpy as jnp
from jax import lax
from jax.experimental import pallas as pl
from jax.experimental.pallas import tpu as pltpu
from jax.experimental.pallas import tpu_sc as plsc

_NC = 2
_NS = 16
_NW = _NC * _NS
_L = 16
_CH = 128

_IDX0 = 124


def kernel(x):
    n, d = x.shape
    assert n % (_NW * _CH) == 0
    rows = n // _NW
    nch = rows // _CH
    mesh = plsc.VectorSubcoreMesh(core_axis_name="c", subcore_axis_name="s")

    @functools.partial(
        pl.kernel,
        out_type=jax.ShapeDtypeStruct((n * d,), x.dtype),
        mesh=mesh,
        scratch_types=[
            pltpu.VMEM((rows * d,), jnp.float32),
            pltpu.VMEM((4 * nch, _CH), jnp.int32),
            pltpu.VMEM((rows,), jnp.float32),
            pltpu.VMEM((rows,), jnp.float32),
            pltpu.VMEM((rows,), jnp.float32),
            pltpu.VMEM((rows,), jnp.float32),
            pltpu.SemaphoreType.DMA,
            pltpu.SemaphoreType.DMA,
        ],
    )
    def run(x_hbm, out_hbm, slab, idx, bb, cb, lb, gb, ssem, gsem):
        wid = lax.axis_index("s") * _NC + lax.axis_index("c")
        base = wid * rows
        flat_sl = pl.ds(base * d, rows * d)
        x_slab = x_hbm.at[flat_sl]

        in_cp = pltpu.async_copy(x_slab, slab, ssem)

        lane = lax.iota(jnp.int32, _L)
        for j in range(4):
            for ch in range(nch):
                for v in range(_CH // _L):
                    row0 = ch * _CH + v * _L
                    idx[j * nch + ch, pl.ds(v * _L, _L)] = (
                        (row0 + lane) * d + (_IDX0 + j)
                    )

        cps = []
        for j, buf in ((0, bb), (1, cb), (2, lb), (3, gb)):
            for ch in range(nch):
                cps.append(pltpu.async_copy(
                    x_slab.at[idx.at[j * nch + ch]],
                    buf.at[pl.ds(ch * _CH, _CH)], gsem))
        for cp in cps:
            cp.wait()

        def step(k, carry):
            sl = pl.ds(k * _L, _L)
            b = bb[sl]
            c = cb[sl]
            l = lb[sl]
            g = gb[sl]
            cb[sl] = c / (c + b)
            lb[sl] = c / (c + l + g)
            gb[sl] = g / (g + l)
            return carry

        lax.fori_loop(0, rows // _L, step, 0)

        in_cp.wait()
        pltpu.sync_copy(slab, out_hbm.at[flat_sl])
        out_slab = out_hbm.at[flat_sl]
        cps = []
        for j, buf in ((1, cb), (2, lb), (3, gb)):
            for ch in range(nch):
                cps.append(pltpu.async_copy(
                    buf.at[pl.ds(ch * _CH, _CH)],
                    out_slab.at[idx.at[j * nch + ch]], gsem))
        for cp in cps:
            cp.wait()

    return run(x.reshape(-1)).reshape(n, d)


# speedup vs baseline: 2.5609x; 2.3041x over previous
import functools

import jax
import jax.numpy as jnp
from jax import lax
from jax.experimental import pallas as pl
from jax.experimental.pallas import tpu as pltpu
from jax.experimental.pallas import tpu_sc as plsc

_NC = 2
_NS = 16
_NW = _NC * _NS
_L = 16

_IDX0 = 124


def kernel(x):
    n, d = x.shape
    assert n % _NW == 0 and d >= _L
    rows = n // _NW
    mesh = plsc.VectorSubcoreMesh(core_axis_name="c", subcore_axis_name="s")

    ident = list(range(_L))
    dn = lax.GatherDimensionNumbers(
        offset_dims=(), collapsed_slice_dims=(0,), start_index_map=(0,))

    def perm(v, idx):
        return lax.gather(v, idx, dn, slice_sizes=(1,),
                          mode=lax.GatherScatterMode.PROMISE_IN_BOUNDS)

    @functools.partial(
        pl.kernel,
        out_type=jax.ShapeDtypeStruct((n * d,), x.dtype),
        mesh=mesh,
        scratch_types=[
            pltpu.VMEM((rows * d,), jnp.float32),
        ],
    )
    def run(x_hbm, out_hbm, slab):
        wid = lax.axis_index("s") * _NC + lax.axis_index("c")
        base = wid * rows
        flat_sl = pl.ds(base * d, rows * d)
        pltpu.sync_copy(x_hbm.at[flat_sl], slab)

        lane = lax.iota(jnp.int32, _L)
        m13 = lane >= 13
        m14 = lane == 14
        zero = jnp.zeros((_L,), jnp.float32)
        one = jnp.ones((_L,), jnp.float32)
        izero = jnp.zeros((_L,), jnp.int32)
        ione = izero + 1
        e13 = jnp.where(lane == 13, ione, izero)
        e14 = jnp.where(m14, ione, izero)
        e15 = jnp.where(lane == 15, ione, izero)
        i_num = (lane - e14)[:, None]
        i_d2 = (lane - e13 - e15)[:, None]
        i_d3 = (lane + e14)[:, None]

        def step(r, carry):
            off = r * d + (d - _L)
            v = slab[pl.ds(off, _L)]
            num = perm(v, i_num)
            d2 = perm(v, i_d2)
            d3 = perm(v, i_d3)
            den = num + d2 + jnp.where(m14, d3, zero)
            slab[pl.ds(off, _L)] = (
                jnp.where(m13, num, v) / jnp.where(m13, den, one)
            )
            return carry

        lax.fori_loop(0, rows, step, 0)
        pltpu.sync_copy(slab, out_hbm.at[flat_sl])

    return run(x.reshape(-1)).reshape(n, d)


# speedup vs baseline: 2.7000x; 1.0543x over previous
import functools

import jax
import jax.numpy as jnp
from jax import lax
from jax.experimental import pallas as pl
from jax.experimental.pallas import tpu as pltpu
from jax.experimental.pallas import tpu_sc as plsc

_NC = 2
_NS = 16
_NW = _NC * _NS
_L = 16
_NCHK = 4


def kernel(x):
    n, d = x.shape
    assert n % (_NW * _NCHK) == 0 and d >= _L
    rows = n // _NW
    crows = rows // _NCHK
    mesh = plsc.VectorSubcoreMesh(core_axis_name="c", subcore_axis_name="s")

    dn = lax.GatherDimensionNumbers(
        offset_dims=(), collapsed_slice_dims=(0,), start_index_map=(0,))

    def perm(v, idx):
        return lax.gather(v, idx, dn, slice_sizes=(1,),
                          mode=lax.GatherScatterMode.PROMISE_IN_BOUNDS)

    @functools.partial(
        pl.kernel,
        out_type=jax.ShapeDtypeStruct((n * d,), x.dtype),
        mesh=mesh,
        scratch_types=[pltpu.VMEM((rows * d,), jnp.float32)]
        + [pltpu.SemaphoreType.DMA] * (2 * _NCHK),
    )
    def run(x_hbm, out_hbm, slab, *sems):
        wid = lax.axis_index("s") * _NC + lax.axis_index("c")
        base = wid * rows

        in_cps = []
        for k in range(_NCHK):
            hbm_sl = pl.ds((base + k * crows) * d, crows * d)
            loc_sl = pl.ds(k * crows * d, crows * d)
            in_cps.append(
                pltpu.async_copy(x_hbm.at[hbm_sl], slab.at[loc_sl], sems[k]))

        lane = lax.iota(jnp.int32, _L)
        m13 = lane >= 13
        m14 = lane == 14
        zero = jnp.zeros((_L,), jnp.float32)
        one = jnp.ones((_L,), jnp.float32)
        izero = jnp.zeros((_L,), jnp.int32)
        ione = izero + 1
        e13 = jnp.where(lane == 13, ione, izero)
        e14 = jnp.where(m14, ione, izero)
        e15 = jnp.where(lane == 15, ione, izero)
        i_num = (lane - e14)[:, None]
        i_d2 = (lane - e13 - e15)[:, None]
        i_d3 = (lane + e14)[:, None]

        out_cps = []
        for k in range(_NCHK):
            in_cps[k].wait()

            def step(r, carry, k=k):
                off = (k * crows + r) * d + (d - _L)
                v = slab[pl.ds(off, _L)]
                num = perm(v, i_num)
                d2 = perm(v, i_d2)
                d3 = perm(v, i_d3)
                den = num + d2 + jnp.where(m14, d3, zero)
                slab[pl.ds(off, _L)] = (
                    jnp.where(m13, num, v) / jnp.where(m13, den, one)
                )
                return carry

            lax.fori_loop(0, crows, step, 0)
            hbm_sl = pl.ds((base + k * crows) * d, crows * d)
            loc_sl = pl.ds(k * crows * d, crows * d)
            out_cps.append(
                pltpu.async_copy(slab.at[loc_sl], out_hbm.at[hbm_sl],
                                 sems[_NCHK + k]))
        for cp in out_cps:
            cp.wait()

    return run(x.reshape(-1)).reshape(n, d)


# speedup vs baseline: 3.1558x; 1.1688x over previous
import functools

import jax
import jax.numpy as jnp
from jax import lax
from jax.experimental import pallas as pl
from jax.experimental.pallas import tpu as pltpu
from jax.experimental.pallas import tpu_sc as plsc

_NC = 2
_NS = 16
_NW = _NC * _NS
_L = 16
_NCHK = 8


def kernel(x):
    n, d = x.shape
    assert n % (_NW * _NCHK) == 0 and d >= _L
    rows = n // _NW
    crows = rows // _NCHK
    mesh = plsc.VectorSubcoreMesh(core_axis_name="c", subcore_axis_name="s")

    dn = lax.GatherDimensionNumbers(
        offset_dims=(), collapsed_slice_dims=(0,), start_index_map=(0,))

    def perm(v, idx):
        return lax.gather(v, idx, dn, slice_sizes=(1,),
                          mode=lax.GatherScatterMode.PROMISE_IN_BOUNDS)

    @functools.partial(
        pl.kernel,
        out_type=jax.ShapeDtypeStruct((n * d,), x.dtype),
        mesh=mesh,
        scratch_types=[pltpu.VMEM((rows * d,), jnp.float32)]
        + [pltpu.SemaphoreType.DMA] * (2 * _NCHK),
    )
    def run(x_hbm, out_hbm, slab, *sems):
        wid = lax.axis_index("s") * _NC + lax.axis_index("c")
        base = wid * rows

        in_cps = []
        for k in range(_NCHK):
            hbm_sl = pl.ds((base + k * crows) * d, crows * d)
            loc_sl = pl.ds(k * crows * d, crows * d)
            in_cps.append(
                pltpu.async_copy(x_hbm.at[hbm_sl], slab.at[loc_sl], sems[k]))

        lane = lax.iota(jnp.int32, _L)
        m13 = lane >= 13
        m14 = lane == 14
        zero = jnp.zeros((_L,), jnp.float32)
        one = jnp.ones((_L,), jnp.float32)
        izero = jnp.zeros((_L,), jnp.int32)
        ione = izero + 1
        e13 = jnp.where(lane == 13, ione, izero)
        e14 = jnp.where(m14, ione, izero)
        e15 = jnp.where(lane == 15, ione, izero)
        i_num = (lane - e14)[:, None]
        i_d2 = (lane - e13 - e15)[:, None]
        i_d3 = (lane + e14)[:, None]

        out_cps = []
        for k in range(_NCHK):
            in_cps[k].wait()

            def step(r, carry, k=k):
                for u in range(4):
                    off = (k * crows + r * 4 + u) * d + (d - _L)
                    v = slab[pl.ds(off, _L)]
                    num = perm(v, i_num)
                    d2 = perm(v, i_d2)
                    d3 = perm(v, i_d3)
                    den = num + d2 + jnp.where(m14, d3, zero)
                    slab[pl.ds(off, _L)] = (
                        jnp.where(m13, num, v) / jnp.where(m13, den, one)
                    )
                return carry

            lax.fori_loop(0, crows // 4, step, 0)
            hbm_sl = pl.ds((base + k * crows) * d, crows * d)
            loc_sl = pl.ds(k * crows * d, crows * d)
            out_cps.append(
                pltpu.async_copy(slab.at[loc_sl], out_hbm.at[hbm_sl],
                                 sems[_NCHK + k]))
        for cp in out_cps:
            cp.wait()

    return run(x.reshape(-1)).reshape(n, d)
